# Initial kernel scaffold; baseline (speedup 1.0000x reference)
import jax
import jax.numpy as jnp
from jax.experimental import pallas as pl


def kernel(xyz, points, W0_0, b0_0, W0_1, b0_1, W1_0, b1_0, W1_1, b1_1, W2_0, b2_0, W2_1, b2_1):
    raise NotImplementedError("write your pallas kernel here")


# speedup vs baseline: 1.3014x; 1.3014x over previous
import functools

import jax
import jax.numpy as jnp
from jax import lax
from jax.experimental import pallas as pl
from jax.experimental.pallas import tpu as pltpu

B, N, C = 2, 16384, 16
NPOINT = 1024
RADIUS_LIST = [0.1, 0.2, 0.4]
NSAMPLE_LIST = [16, 32, 64]
NROW = 128


def _fps_body(xt_ref, idx_ref, dist_ref):
    row = lax.broadcasted_iota(jnp.int32, (NROW, 128), 0)
    col = lax.broadcasted_iota(jnp.int32, (NROW, 128), 1)
    iota = row * 128 + col
    dist_ref[...] = jnp.full((B, NROW, 128), 1e10, dtype=jnp.float32)

    def body(i, fars):
        new_fars = []
        for b in range(B):
            far = fars[b]
            idx_ref[b, i] = far
            x = xt_ref[b, 0]
            y = xt_ref[b, 1]
            z = xt_ref[b, 2]
            sel = iota == far
            zero = jnp.zeros((), jnp.float32)
            cx = jnp.sum(jnp.where(sel, x, zero))
            cy = jnp.sum(jnp.where(sel, y, zero))
            cz = jnp.sum(jnp.where(sel, z, zero))
            dx = x - cx
            dy = y - cy
            dz = z - cz
            d = dx * dx + dy * dy + dz * dz
            dmin = jnp.minimum(dist_ref[b], d)
            dist_ref[b] = dmin
            m = jnp.max(dmin)
            far_n = jnp.min(jnp.where(dmin == m, iota, jnp.int32(2**31 - 1)))
            new_fars.append(far_n)
        return tuple(new_fars)

    lax.fori_loop(0, NPOINT, body, tuple(jnp.int32(0) for _ in range(B)))


def _fps(xyz):
    xt = xyz.transpose(0, 2, 1).reshape(B, 3, NROW, 128)
    fps_idx = pl.pallas_call(
        _fps_body,
        out_shape=jax.ShapeDtypeStruct((B, NPOINT), jnp.int32),
        in_specs=[pl.BlockSpec(memory_space=pltpu.VMEM)],
        out_specs=pl.BlockSpec(memory_space=pltpu.SMEM),
        scratch_shapes=[pltpu.VMEM((B, NROW, 128), jnp.float32)],
    )(xt)
    return fps_idx


def _square_distance(a, b):
    aa = jnp.sum(a * a, axis=-1)[:, :, None]
    bb = jnp.sum(b * b, axis=-1)[:, None, :]
    ab = jnp.einsum('bsd,bnd->bsn', a, b)
    return aa + bb - 2.0 * ab


def _query_ball_point(radius, nsample, xyz, new_xyz):
    n = xyz.shape[1]
    sqrdists = _square_distance(new_xyz, xyz)
    mask = sqrdists <= radius ** 2
    idx = jnp.where(mask, jnp.arange(n, dtype=jnp.int32)[None, None, :], n)
    idx = jnp.sort(idx, axis=-1)[:, :, :nsample]
    first = idx[:, :, :1]
    idx = jnp.where(idx == n, first, idx)
    return idx


def _group_point(points, idx):
    return jax.vmap(lambda p, i: p[i])(points, idx)


def kernel(xyz, points, W0_0, b0_0, W0_1, b0_1, W1_0, b1_0, W1_1, b1_1,
           W2_0, b2_0, W2_1, b2_1):
    params = [[(W0_0, b0_0), (W0_1, b0_1)], [(W1_0, b1_0), (W1_1, b1_1)],
              [(W2_0, b2_0), (W2_1, b2_1)]]
    fps_idx = _fps(xyz)
    new_xyz = jnp.take_along_axis(xyz, fps_idx[:, :, None], axis=1)
    new_points_list = []
    for i in range(3):
        radius = RADIUS_LIST[i]
        nsample = NSAMPLE_LIST[i]
        idx = _query_ball_point(radius, nsample, xyz, new_xyz)
        grouped_xyz = _group_point(xyz, idx)
        grouped_xyz = grouped_xyz - new_xyz[:, :, None, :]
        grouped_points = _group_point(points, idx)
        grouped_points = jnp.concatenate([grouped_points, grouped_xyz], axis=-1)
        for (W, b) in params[i]:
            grouped_points = jax.nn.relu(grouped_points @ W + b)
        new_points = jnp.max(grouped_points, axis=2)
        new_points_list.append(new_points)
    new_points_concat = jnp.concatenate(new_points_list, axis=-1)
    return (new_xyz, new_points_concat)


# speedup vs baseline: 1.6252x; 1.2488x over previous
import functools

import jax
import jax.numpy as jnp
from jax import lax
from jax.experimental import pallas as pl
from jax.experimental.pallas import tpu as pltpu
from jax.experimental.pallas import tpu_sc as plsc

B, N, C = 2, 16384, 16
NPOINT = 1024
RADIUS_LIST = [0.1, 0.2, 0.4]
NSAMPLE_LIST = [16, 32, 64]
NROW = 128
NS_TOTAL = sum(NSAMPLE_LIST)
IDX_TOTAL = B * NPOINT * NS_TOTAL
NWORK = 32
ROWS_PER_W = IDX_TOTAL // NWORK
GCHUNK = 128
NCHUNK = ROWS_PER_W // GCHUNK
DPAD = 32


def _fps_body(xt_ref, idx_ref, nxyz_ref, dist_ref):
    row = lax.broadcasted_iota(jnp.int32, (NROW, 128), 0)
    col = lax.broadcasted_iota(jnp.int32, (NROW, 128), 1)
    iota = row * 128 + col
    dist_ref[...] = jnp.full((B, NROW, 128), 1e10, dtype=jnp.float32)

    def body(i, fars):
        new_fars = []
        for b in range(B):
            far = fars[b]
            idx_ref[b, i] = far
            x = xt_ref[b, 0]
            y = xt_ref[b, 1]
            z = xt_ref[b, 2]
            sel = iota == far
            zero = jnp.zeros((), jnp.float32)
            cx = jnp.sum(jnp.where(sel, x, zero))
            cy = jnp.sum(jnp.where(sel, y, zero))
            cz = jnp.sum(jnp.where(sel, z, zero))
            nxyz_ref[0, b, i] = cx
            nxyz_ref[1, b, i] = cy
            nxyz_ref[2, b, i] = cz
            dx = x - cx
            dy = y - cy
            dz = z - cz
            d = dx * dx + dy * dy + dz * dz
            dmin = jnp.minimum(dist_ref[b], d)
            dist_ref[b] = dmin
            m = jnp.max(dmin)
            far_n = jnp.min(jnp.where(dmin == m, iota, jnp.int32(2**31 - 1)))
            new_fars.append(far_n)
        return tuple(new_fars)

    lax.fori_loop(0, NPOINT, body, tuple(jnp.int32(0) for _ in range(B)))


def _fps(xyz):
    xt = xyz.transpose(0, 2, 1).reshape(B, 3, NROW, 128)
    fps_idx, new_xyz = pl.pallas_call(
        _fps_body,
        out_shape=[
            jax.ShapeDtypeStruct((B, NPOINT), jnp.int32),
            jax.ShapeDtypeStruct((3, B, NPOINT), jnp.float32),
        ],
        in_specs=[pl.BlockSpec(memory_space=pltpu.VMEM)],
        out_specs=[
            pl.BlockSpec(memory_space=pltpu.SMEM),
            pl.BlockSpec(memory_space=pltpu.SMEM),
        ],
        scratch_shapes=[pltpu.VMEM((B, NROW, 128), jnp.float32)],
    )(xt)
    return fps_idx, new_xyz.transpose(1, 2, 0)


def _sc_gather(xpad, idx_flat):
    mesh = plsc.VectorSubcoreMesh(core_axis_name="c", subcore_axis_name="s")

    @functools.partial(
        pl.kernel,
        mesh=mesh,
        out_type=jax.ShapeDtypeStruct((IDX_TOTAL, DPAD), jnp.float32),
        scratch_types=[
            pltpu.VMEM((GCHUNK,), jnp.int32),
            pltpu.VMEM((GCHUNK, DPAD), jnp.float32),
            pltpu.SemaphoreType.DMA,
        ],
        compiler_params=pltpu.CompilerParams(use_tc_tiling_on_sc=False),
    )
    def gk(xpad_hbm, idx_hbm, out_hbm, idx_v, rows_v, sem):
        wid = lax.axis_index("s") * 2 + lax.axis_index("c")
        base = wid * ROWS_PER_W

        def chunk(t, carry):
            off = base + t * GCHUNK
            pltpu.sync_copy(idx_hbm.at[pl.ds(off, GCHUNK)], idx_v)
            pltpu.async_copy(xpad_hbm.at[idx_v], rows_v, sem).wait()
            pltpu.sync_copy(rows_v, out_hbm.at[pl.ds(off, GCHUNK)])
            return carry

        lax.fori_loop(0, NCHUNK, chunk, jnp.int32(0))

    return gk(xpad, idx_flat)


def _square_distance(a, b):
    aa = jnp.sum(a * a, axis=-1)[:, :, None]
    bb = jnp.sum(b * b, axis=-1)[:, None, :]
    ab = jnp.einsum('bsd,bnd->bsn', a, b)
    return aa + bb - 2.0 * ab


def _query_ball_point(radius, nsample, xyz, new_xyz):
    n = xyz.shape[1]
    sqrdists = _square_distance(new_xyz, xyz)
    mask = sqrdists <= radius ** 2
    idx = jnp.where(mask, jnp.arange(n, dtype=jnp.int32)[None, None, :], n)
    idx = jnp.sort(idx, axis=-1)[:, :, :nsample]
    first = idx[:, :, :1]
    idx = jnp.where(idx == n, first, idx)
    return idx


def kernel(xyz, points, W0_0, b0_0, W0_1, b0_1, W1_0, b1_0, W1_1, b1_1,
           W2_0, b2_0, W2_1, b2_1):
    params = [[(W0_0, b0_0), (W0_1, b0_1)], [(W1_0, b1_0), (W1_1, b1_1)],
              [(W2_0, b2_0), (W2_1, b2_1)]]
    fps_idx, new_xyz = _fps(xyz)

    zcols = jnp.zeros((B * N, DPAD - C - 3), jnp.float32)
    xpad = jnp.concatenate(
        [points.reshape(B * N, C), xyz.reshape(B * N, 3), zcols], axis=1)
    czero = jnp.zeros((B * NPOINT, C), jnp.float32)
    cpad = jnp.concatenate(
        [czero, new_xyz.reshape(B * NPOINT, 3),
         jnp.zeros((B * NPOINT, DPAD - C - 3), jnp.float32)], axis=1)

    boff = (jnp.arange(B, dtype=jnp.int32) * N)[:, None, None]
    idx_parts = []
    for i in range(3):
        idx = _query_ball_point(RADIUS_LIST[i], NSAMPLE_LIST[i], xyz, new_xyz)
        idx_parts.append((idx + boff).reshape(-1))
    idx_flat = jnp.concatenate(idx_parts)

    xg = _sc_gather(xpad, idx_flat)

    outs = []
    off = 0
    for i in range(3):
        ns = NSAMPLE_LIST[i]
        rows = B * NPOINT * ns
        xs = xg[off:off + rows].reshape(B * NPOINT, ns, DPAD)
        off += rows
        xs = xs - cpad[:, None, :]
        (W1, b1), (W2, b2) = params[i]
        w1pad = jnp.concatenate(
            [W1, jnp.zeros((DPAD - C - 3, W1.shape[1]), jnp.float32)], axis=0)
        h = jax.nn.relu(xs @ w1pad + b1)
        h = jax.nn.relu(h @ W2 + b2)
        outs.append(jnp.max(h, axis=1).reshape(B, NPOINT, -1))
    new_points_concat = jnp.concatenate(outs, axis=-1)
    return (new_xyz, new_points_concat)


# speedup vs baseline: 23.3024x; 14.3382x over previous
import functools

import jax
import jax.numpy as jnp
from jax import lax
from jax.experimental import pallas as pl
from jax.experimental.pallas import tpu as pltpu
from jax.experimental.pallas import tpu_sc as plsc

B, N, C = 2, 16384, 16
NPOINT = 1024
RADIUS_LIST = [0.1, 0.2, 0.4]
NSAMPLE_LIST = [16, 32, 64]
NROW = 128
NS_TOTAL = sum(NSAMPLE_LIST)
IDX_TOTAL = B * NPOINT * NS_TOTAL
NWORK = 32
ROWS_PER_W = IDX_TOTAL // NWORK
GCHUNK = 128
NCHUNK = ROWS_PER_W // GCHUNK
DPAD = 32


def _fps_body(xt_ref, idx_ref, nxyz_ref, dist_ref):
    row = lax.broadcasted_iota(jnp.int32, (NROW, 128), 0)
    col = lax.broadcasted_iota(jnp.int32, (NROW, 128), 1)
    iota = row * 128 + col
    dist_ref[...] = jnp.full((B, NROW, 128), 1e10, dtype=jnp.float32)

    def body(i, fars):
        new_fars = []
        for b in range(B):
            far = fars[b]
            idx_ref[b, i] = far
            x = xt_ref[b, 0]
            y = xt_ref[b, 1]
            z = xt_ref[b, 2]
            sel = iota == far
            zero = jnp.zeros((), jnp.float32)
            cx = jnp.sum(jnp.where(sel, x, zero))
            cy = jnp.sum(jnp.where(sel, y, zero))
            cz = jnp.sum(jnp.where(sel, z, zero))
            nxyz_ref[0, b, i] = cx
            nxyz_ref[1, b, i] = cy
            nxyz_ref[2, b, i] = cz
            dx = x - cx
            dy = y - cy
            dz = z - cz
            d = dx * dx + dy * dy + dz * dz
            dmin = jnp.minimum(dist_ref[b], d)
            dist_ref[b] = dmin
            m = jnp.max(dmin)
            far_n = jnp.min(jnp.where(dmin == m, iota, jnp.int32(2**31 - 1)))
            new_fars.append(far_n)
        return tuple(new_fars)

    lax.fori_loop(0, NPOINT, body, tuple(jnp.int32(0) for _ in range(B)))


def _fps(xyz):
    xt = xyz.transpose(0, 2, 1).reshape(B, 3, NROW, 128)
    fps_idx, new_xyz = pl.pallas_call(
        _fps_body,
        out_shape=[
            jax.ShapeDtypeStruct((B, NPOINT), jnp.int32),
            jax.ShapeDtypeStruct((3, B, NPOINT), jnp.float32),
        ],
        in_specs=[pl.BlockSpec(memory_space=pltpu.VMEM)],
        out_specs=[
            pl.BlockSpec(memory_space=pltpu.SMEM),
            pl.BlockSpec(memory_space=pltpu.SMEM),
        ],
        scratch_shapes=[pltpu.VMEM((B, NROW, 128), jnp.float32)],
    )(xt)
    return fps_idx, new_xyz.transpose(1, 2, 0)


def _sc_gather(xpad, idx_flat):
    mesh = plsc.VectorSubcoreMesh(core_axis_name="c", subcore_axis_name="s")

    @functools.partial(
        pl.kernel,
        mesh=mesh,
        out_type=jax.ShapeDtypeStruct((IDX_TOTAL, DPAD), jnp.float32),
        scratch_types=[
            pltpu.VMEM((GCHUNK,), jnp.int32),
            pltpu.VMEM((GCHUNK, DPAD), jnp.float32),
            pltpu.SemaphoreType.DMA,
        ],
        compiler_params=pltpu.CompilerParams(use_tc_tiling_on_sc=False),
    )
    def gk(xpad_hbm, idx_hbm, out_hbm, idx_v, rows_v, sem):
        wid = lax.axis_index("s") * 2 + lax.axis_index("c")
        base = wid * ROWS_PER_W

        def chunk(t, carry):
            off = base + t * GCHUNK
            pltpu.sync_copy(idx_hbm.at[pl.ds(off, GCHUNK)], idx_v)
            pltpu.async_copy(xpad_hbm.at[idx_v], rows_v, sem).wait()
            pltpu.sync_copy(rows_v, out_hbm.at[pl.ds(off, GCHUNK)])
            return carry

        lax.fori_loop(0, NCHUNK, chunk, jnp.int32(0))

    return gk(xpad, idx_flat)


GD = 128


def _sqd_body(nx_ref, xt_ref, d_ref):
    nx = nx_ref[0]
    xt = xt_ref[0]
    aa = jnp.sum(nx * nx, axis=1, keepdims=True)
    bb = (xt[0] * xt[0] + xt[1] * xt[1] + xt[2] * xt[2])[None, :]
    ab = jnp.dot(nx, xt, preferred_element_type=jnp.float32)
    d_ref[0] = aa + bb - 2.0 * ab


def _sqdists(new_xyz, xyzT):
    return pl.pallas_call(
        _sqd_body,
        grid=(B, NPOINT // GD),
        out_shape=jax.ShapeDtypeStruct((B, NPOINT, N), jnp.float32),
        in_specs=[
            pl.BlockSpec((1, GD, 3), lambda b, j: (b, j, 0)),
            pl.BlockSpec((1, 3, N), lambda b, j: (b, 0, 0)),
        ],
        out_specs=pl.BlockSpec((1, GD, N), lambda b, j: (b, j, 0)),
    )(new_xyz, xyzT)


CPW = B * NPOINT // NWORK
STRIDES = [ns + 16 for ns in NSAMPLE_LIST]
NCHK = N // 16


def _sc_ballquery(d_flat):
    mesh = plsc.VectorSubcoreMesh(core_axis_name="c", subcore_axis_name="s")
    r2s = [r * r for r in RADIUS_LIST]

    @functools.partial(
        pl.kernel,
        mesh=mesh,
        out_type=[
            jax.ShapeDtypeStruct((B * NPOINT * ns,), jnp.int32)
            for ns in NSAMPLE_LIST
        ],
        scratch_types=[
            pltpu.VMEM((2 * N,), jnp.float32),
            pltpu.VMEM((CPW * STRIDES[0] + 16,), jnp.int32),
            pltpu.VMEM((CPW * STRIDES[1] + 16,), jnp.int32),
            pltpu.VMEM((CPW * STRIDES[2] + 16,), jnp.int32),
            pltpu.SemaphoreType.DMA,
        ],
        compiler_params=pltpu.CompilerParams(
            use_tc_tiling_on_sc=False, needs_layout_passes=False),
    )
    def bq(d_hbm, o0_hbm, o1_hbm, o2_hbm, dbuf, buf0, buf1, buf2, sem):
        wid = lax.axis_index("s") * 2 + lax.axis_index("c")
        b = wid // 16
        base_pt = b * N
        row0 = wid * CPW
        lane = lax.broadcasted_iota(jnp.int32, (16,), 0)
        bufs = (buf0, buf1, buf2)
        outs = (o0_hbm, o1_hbm, o2_hbm)
        pltpu.async_copy(d_hbm.at[pl.ds(row0 * N, N)],
                         dbuf.at[pl.ds(0, N)], sem)

        def per_centroid(ci, carry):
            pltpu.make_async_copy(d_hbm.at[pl.ds(0, N)],
                                  dbuf.at[pl.ds(0, N)], sem).wait()
            nxt = jnp.minimum(ci + 1, CPW - 1)
            pltpu.async_copy(
                d_hbm.at[pl.ds((row0 + nxt) * N, N)],
                dbuf.at[pl.ds(((ci + 1) % 2) * N, N)], sem)
            pbase = (ci % 2) * N

            def cond(st):
                t, o0, o1, o2 = st
                return (t < NCHK) & ((o0 < NSAMPLE_LIST[0])
                                     | (o1 < NSAMPLE_LIST[1])
                                     | (o2 < NSAMPLE_LIST[2]))

            def body(st):
                t, o0, o1, o2 = st
                off = t * 16
                d = dbuf[pl.ds(pbase + off, 16)]
                gi = lane + (off + base_pt)
                os_ = [o0, o1, o2]
                new_os = []
                for r in range(3):
                    m = d <= r2s[r]
                    pc = plsc.cumsum(m.astype(jnp.int32))
                    pos = ci * STRIDES[r] + jnp.minimum(
                        os_[r], NSAMPLE_LIST[r])
                    trash = jnp.int32(CPW * STRIDES[r])
                    tgt = jnp.where(m, pos + pc - 1, trash)
                    plsc.store_scatter(bufs[r], [tgt], gi)
                    new_os.append(os_[r] + pc[15])
                return (t + 1, new_os[0], new_os[1], new_os[2])

            z32 = jnp.int32(0)
            _, o0, o1, o2 = lax.while_loop(cond, body, (z32, z32, z32, z32))
            os_ = [o0, o1, o2]
            for r in range(3):
                ns = NSAMPLE_LIST[r]
                stride = STRIDES[r]
                cnt = jnp.minimum(os_[r], ns)
                first = bufs[r][pl.ds(ci * stride, 16)][0]
                for k in range(ns // 16):
                    sl = pl.ds(ci * stride + k * 16, 16)
                    v = bufs[r][sl]
                    vfix = jnp.where(lane + (k * 16) < cnt, v, first)
                    bufs[r][sl] = vfix
                pltpu.sync_copy(
                    bufs[r].at[pl.ds(ci * stride, ns)],
                    outs[r].at[pl.ds((wid * CPW + ci) * ns, ns)])
            return carry

        lax.fori_loop(0, CPW, per_centroid, jnp.int32(0))
        pltpu.make_async_copy(d_hbm.at[pl.ds(0, N)],
                              dbuf.at[pl.ds(0, N)], sem).wait()

    return bq(d_flat)


def kernel(xyz, points, W0_0, b0_0, W0_1, b0_1, W1_0, b1_0, W1_1, b1_1,
           W2_0, b2_0, W2_1, b2_1):
    params = [[(W0_0, b0_0), (W0_1, b0_1)], [(W1_0, b1_0), (W1_1, b1_1)],
              [(W2_0, b2_0), (W2_1, b2_1)]]
    fps_idx, new_xyz = _fps(xyz)

    zcols = jnp.zeros((B * N, DPAD - C - 3), jnp.float32)
    xpad = jnp.concatenate(
        [points.reshape(B * N, C), xyz.reshape(B * N, 3), zcols], axis=1)
    czero = jnp.zeros((B * NPOINT, C), jnp.float32)
    cpad = jnp.concatenate(
        [czero, new_xyz.reshape(B * NPOINT, 3),
         jnp.zeros((B * NPOINT, DPAD - C - 3), jnp.float32)], axis=1)

    sq = _sqdists(new_xyz, xyz.transpose(0, 2, 1))
    idx_parts = _sc_ballquery(sq.reshape(-1))
    idx_flat = jnp.concatenate(idx_parts)

    xg = _sc_gather(xpad, idx_flat)

    outs = []
    off = 0
    for i in range(3):
        ns = NSAMPLE_LIST[i]
        rows = B * NPOINT * ns
        xs = xg[off:off + rows].reshape(B * NPOINT, ns, DPAD)
        off += rows
        xs = xs - cpad[:, None, :]
        (W1, b1), (W2, b2) = params[i]
        w1pad = jnp.concatenate(
            [W1, jnp.zeros((DPAD - C - 3, W1.shape[1]), jnp.float32)], axis=0)
        h = jax.nn.relu(xs @ w1pad + b1)
        h = jax.nn.relu(h @ W2 + b2)
        outs.append(jnp.max(h, axis=1).reshape(B, NPOINT, -1))
    new_points_concat = jnp.concatenate(outs, axis=-1)
    return (new_xyz, new_points_concat)
